# baseline (speedup 1.0000x reference)
import functools

import jax
import jax.numpy as jnp
from jax import lax
from jax.experimental import pallas as pl
from jax.experimental.pallas import tpu as pltpu
from jax.experimental.pallas import tpu_sc as plsc

EMB = 16
VOCABS = (5, 6, 2)
NCODES = VOCABS[0] * VOCABS[1] * VOCABS[2]
LANES = 16


@functools.cache
def _build_sc_kernel(E: int):
    NC, NS = 2, 16
    NW = NC * NS
    rows_per_w = E // NW
    CHUNK = 2000
    assert rows_per_w % CHUNK == 0
    nchunks = rows_per_w // CHUNK

    mesh = plsc.VectorSubcoreMesh(core_axis_name="c", subcore_axis_name="s")

    @functools.partial(
        pl.kernel,
        out_type=jax.ShapeDtypeStruct((E, EMB), jnp.float32),
        mesh=mesh,
        compiler_params=pltpu.CompilerParams(
            needs_layout_passes=False, use_tc_tiling_on_sc=False),
        scratch_types=[
            pltpu.VMEM((VOCABS[0], EMB), jnp.float32),
            pltpu.VMEM((VOCABS[1], EMB), jnp.float32),
            pltpu.VMEM((VOCABS[2], EMB), jnp.float32),
            pltpu.VMEM((NCODES, EMB), jnp.float32),
            pltpu.VMEM((CHUNK, 3), jnp.int32),
            pltpu.VMEM((CHUNK, EMB), jnp.float32),
        ],
    )
    def body(edge_hbm, w0_hbm, w1_hbm, w2_hbm, out_hbm,
             w0_v, w1_v, w2_v, table_v, in_v, out_v):
        wid = lax.axis_index("s") * NC + lax.axis_index("c")
        my_base = wid * rows_per_w

        pltpu.sync_copy(w0_hbm, w0_v)
        pltpu.sync_copy(w1_hbm, w1_v)
        pltpu.sync_copy(w2_hbm, w2_v)
        for a in range(VOCABS[0]):
            for b in range(VOCABS[1]):
                ab = w0_v[a] + w1_v[b]
                for c in range(VOCABS[2]):
                    table_v[(a * VOCABS[1] + b) * VOCABS[2] + c] = ab + w2_v[c]

        iota = lax.iota(jnp.int32, LANES)
        col0 = jnp.zeros((LANES,), jnp.int32)
        col1 = jnp.full((LANES,), 1, jnp.int32)
        col2 = jnp.full((LANES,), 2, jnp.int32)

        @pl.loop(0, nchunks)
        def _chunk(g):
            base = my_base + g * CHUNK
            pltpu.sync_copy(edge_hbm.at[pl.ds(base, CHUNK)], in_v)

            @pl.loop(0, CHUNK // LANES)
            def _grp(j):
                rows = j * LANES + iota
                a = plsc.load_gather(in_v, [rows, col0])
                b = plsc.load_gather(in_v, [rows, col1])
                c = plsc.load_gather(in_v, [rows, col2])
                code = (a * VOCABS[1] + b) * VOCABS[2] + c
                for d in range(EMB):
                    dv = jnp.full((LANES,), d, jnp.int32)
                    v = plsc.load_gather(table_v, [code, dv])
                    plsc.store_scatter(out_v, [rows, dv], v)

            pltpu.sync_copy(out_v, out_hbm.at[pl.ds(base, CHUNK)])

    return body


def kernel(edge_attr, W0, W1, W2):
    edge_attr = edge_attr.astype(jnp.int32)
    E = edge_attr.shape[0]
    return _build_sc_kernel(E)(edge_attr, W0, W1, W2)


# speedup vs baseline: 8.0419x; 8.0419x over previous
import functools

import jax
import jax.numpy as jnp
from jax import lax
from jax.experimental import pallas as pl
from jax.experimental.pallas import tpu as pltpu
from jax.experimental.pallas import tpu_sc as plsc

EMB = 16
VOCABS = (5, 6, 2)
NCODES = VOCABS[0] * VOCABS[1] * VOCABS[2]
LANES = 16
CHUNKT = 20
CHUNK = CHUNKT * 128
TILE_W = 8 * 128


@functools.cache
def _build_sc_kernel(E: int):
    NC, NS = 2, 16
    NW = NC * NS
    ntile = E // 128
    assert E % 128 == 0 and ntile % CHUNKT == 0
    nchunk = ntile // CHUNKT

    mesh = plsc.VectorSubcoreMesh(core_axis_name="c", subcore_axis_name="s")

    @functools.partial(
        pl.kernel,
        out_type=jax.ShapeDtypeStruct((2, ntile * TILE_W), jnp.float32),
        mesh=mesh,
        compiler_params=pltpu.CompilerParams(
            needs_layout_passes=False, use_tc_tiling_on_sc=False),
        scratch_types=[
            pltpu.VMEM((13 * EMB,), jnp.float32),
            pltpu.VMEM((NCODES * EMB,), jnp.float32),
            pltpu.VMEM((3 * CHUNK,), jnp.int32),
            pltpu.VMEM((2 * CHUNKT * TILE_W,), jnp.float32),
        ],
    )
    def body(edge_hbm, w_hbm, out_hbm, w_v, table_v, in_v, out_v):
        wid = lax.axis_index("s") * NC + lax.axis_index("c")

        pltpu.sync_copy(w_hbm, w_v)
        off1 = VOCABS[0] * EMB
        off2 = (VOCABS[0] + VOCABS[1]) * EMB
        for a in range(VOCABS[0]):
            for b in range(VOCABS[1]):
                ab = w_v[pl.ds(a * EMB, EMB)] + w_v[pl.ds(off1 + b * EMB, EMB)]
                for c in range(VOCABS[2]):
                    k = (a * VOCABS[1] + b) * VOCABS[2] + c
                    table_v[pl.ds(k * EMB, EMB)] = ab + w_v[pl.ds(off2 + c * EMB, EMB)]

        nw_chunks = (nchunk - wid + NW - 1) // NW

        @pl.loop(0, nw_chunks)
        def _chunk(i):
            cid = wid + i * NW
            rbase = cid * CHUNK
            for k in range(3):
                pltpu.sync_copy(edge_hbm.at[k, pl.ds(rbase, CHUNK)],
                                in_v.at[pl.ds(k * CHUNK, CHUNK)])

            @pl.loop(0, CHUNK // LANES)
            def _grp(j):
                a = in_v[pl.ds(j * LANES, LANES)]
                b = in_v[pl.ds(CHUNK + j * LANES, LANES)]
                c = in_v[pl.ds(2 * CHUNK + j * LANES, LANES)]
                code16 = ((a * VOCABS[1] + b) * VOCABS[2] + c) * EMB
                obase = (j >> 3) * TILE_W + (j & 7) * LANES
                for d in range(EMB):
                    v = plsc.load_gather(table_v, [code16 + d])
                    dt, ds = d // 8, d % 8
                    off = obase + dt * (CHUNKT * TILE_W) + ds * 128
                    out_v[pl.ds(off, LANES)] = v

            for dt in range(2):
                pltpu.sync_copy(
                    out_v.at[pl.ds(dt * CHUNKT * TILE_W, CHUNKT * TILE_W)],
                    out_hbm.at[dt, pl.ds(cid * CHUNKT * TILE_W, CHUNKT * TILE_W)])

    def run(edge_t, w_packed):
        x = body(edge_t, w_packed)
        return x.reshape(2, ntile, 8, 128).transpose(1, 3, 0, 2).reshape(E, EMB)

    return run


def kernel(edge_attr, W0, W1, W2):
    edge_attr = edge_attr.astype(jnp.int32)
    E = edge_attr.shape[0]
    w_packed = jnp.concatenate(
        [W0.reshape(-1), W1.reshape(-1), W2.reshape(-1)])
    return _build_sc_kernel(E)(edge_attr.T, w_packed)


# speedup vs baseline: 11.3476x; 1.4111x over previous
import functools

import jax
import jax.numpy as jnp
from jax import lax
from jax.experimental import pallas as pl
from jax.experimental.pallas import tpu as pltpu
from jax.experimental.pallas import tpu_sc as plsc

EMB = 16
VOCABS = (5, 6, 2)
NCODES = VOCABS[0] * VOCABS[1] * VOCABS[2]
LANES = 16
CHUNKT = 20
CHUNK = CHUNKT * 128
TILE_W = 8 * 128


@functools.cache
def _build_sc_kernel(E: int):
    NC, NS = 2, 16
    NW = NC * NS
    ntile = E // 128
    assert E % 128 == 0 and ntile % CHUNKT == 0
    nchunk = ntile // CHUNKT

    mesh = plsc.VectorSubcoreMesh(core_axis_name="c", subcore_axis_name="s")

    @functools.partial(
        pl.kernel,
        out_type=jax.ShapeDtypeStruct((2, ntile * TILE_W), jnp.float32),
        mesh=mesh,
        compiler_params=pltpu.CompilerParams(
            needs_layout_passes=False, use_tc_tiling_on_sc=False),
        scratch_types=[
            pltpu.VMEM((13 * EMB,), jnp.float32),
            pltpu.VMEM((NCODES * EMB,), jnp.float32),
            pltpu.VMEM((CHUNK,), jnp.int32),
            pltpu.VMEM((2 * CHUNKT * TILE_W,), jnp.float32),
        ],
    )
    def body(edge_hbm, w_hbm, out_hbm, w_v, table_v, in_v, out_v):
        wid = lax.axis_index("s") * NC + lax.axis_index("c")

        pltpu.sync_copy(w_hbm, w_v)
        off1 = VOCABS[0] * EMB
        off2 = (VOCABS[0] + VOCABS[1]) * EMB
        for a in range(VOCABS[0]):
            for b in range(VOCABS[1]):
                ab = w_v[pl.ds(a * EMB, EMB)] + w_v[pl.ds(off1 + b * EMB, EMB)]
                for c in range(VOCABS[2]):
                    k = (a * VOCABS[1] + b) * VOCABS[2] + c
                    table_v[pl.ds(k * EMB, EMB)] = ab + w_v[pl.ds(off2 + c * EMB, EMB)]

        nw_chunks = (nchunk - wid + NW - 1) // NW

        @pl.loop(0, nw_chunks)
        def _chunk(i):
            cid = wid + i * NW
            rbase = cid * CHUNK
            pltpu.sync_copy(edge_hbm.at[pl.ds(rbase, CHUNK)], in_v)

            @pl.loop(0, CHUNK // LANES)
            def _grp(j):
                v = in_v[pl.ds(j * LANES, LANES)]
                a = v & 0xFF
                b = (v >> 8) & 0xFF
                c = (v >> 16) & 0xFF
                code16 = ((a * VOCABS[1] + b) * VOCABS[2] + c) * EMB
                obase = (j >> 3) * TILE_W + (j & 7) * LANES
                for d in range(EMB):
                    v = plsc.load_gather(table_v, [code16 + d])
                    dt, ds = d // 8, d % 8
                    off = obase + dt * (CHUNKT * TILE_W) + ds * 128
                    out_v[pl.ds(off, LANES)] = v

            for dt in range(2):
                pltpu.sync_copy(
                    out_v.at[pl.ds(dt * CHUNKT * TILE_W, CHUNKT * TILE_W)],
                    out_hbm.at[dt, pl.ds(cid * CHUNKT * TILE_W, CHUNKT * TILE_W)])

    def run(edge_packed, w_packed):
        x = body(edge_packed, w_packed)
        return x.reshape(2, ntile, 8, 128).transpose(1, 3, 0, 2).reshape(E, EMB)

    return run


def kernel(edge_attr, W0, W1, W2):
    E = edge_attr.shape[0]
    edge8 = jnp.concatenate(
        [edge_attr.astype(jnp.int8), jnp.zeros((E, 1), jnp.int8)], axis=1)
    edge_packed = lax.bitcast_convert_type(edge8, jnp.int32)
    w_packed = jnp.concatenate(
        [W0.reshape(-1), W1.reshape(-1), W2.reshape(-1)])
    return _build_sc_kernel(E)(edge_packed, w_packed)


# speedup vs baseline: 12.3269x; 1.0863x over previous
import functools

import jax
import jax.numpy as jnp
from jax import lax
from jax.experimental import pallas as pl
from jax.experimental.pallas import tpu as pltpu
from jax.experimental.pallas import tpu_sc as plsc

EMB = 16
VOCABS = (5, 6, 2)
NCODES = VOCABS[0] * VOCABS[1] * VOCABS[2]
LANES = 16
CHUNKT = 20
CHUNK = CHUNKT * 128
TILE_W = 8 * 128
OUT_W = 2 * CHUNKT * TILE_W


@functools.cache
def _build_sc_kernel(E: int):
    NC, NS = 2, 16
    NW = NC * NS
    ntile = E // 128
    assert E % 128 == 0 and ntile % CHUNKT == 0
    nchunk = ntile // CHUNKT
    niter = -(-nchunk // NW)
    niter += niter % 2

    mesh = plsc.VectorSubcoreMesh(core_axis_name="c", subcore_axis_name="s")

    @functools.partial(
        pl.kernel,
        out_type=jax.ShapeDtypeStruct((2, ntile * TILE_W), jnp.float32),
        mesh=mesh,
        compiler_params=pltpu.CompilerParams(
            needs_layout_passes=False, use_tc_tiling_on_sc=False),
        scratch_types=[
            pltpu.VMEM((13 * EMB,), jnp.float32),
            pltpu.VMEM((NCODES * EMB,), jnp.float32),
            pltpu.VMEM((2 * CHUNK,), jnp.int32),
            pltpu.VMEM((2 * OUT_W,), jnp.float32),
            pltpu.SemaphoreType.DMA,
            pltpu.SemaphoreType.DMA,
            pltpu.SemaphoreType.DMA,
            pltpu.SemaphoreType.DMA,
        ],
    )
    def body(edge_hbm, w_hbm, out_hbm, w_v, table_v, in_v, out_v,
             in_sem0, in_sem1, out_sem0, out_sem1):
        wid = lax.axis_index("s") * NC + lax.axis_index("c")
        in_sems = (in_sem0, in_sem1)
        out_sems = (out_sem0, out_sem1)
        nw_chunks = (nchunk - wid + NW - 1) // NW

        pltpu.sync_copy(w_hbm, w_v)
        off1 = VOCABS[0] * EMB
        off2 = (VOCABS[0] + VOCABS[1]) * EMB
        for a in range(VOCABS[0]):
            for b in range(VOCABS[1]):
                ab = w_v[pl.ds(a * EMB, EMB)] + w_v[pl.ds(off1 + b * EMB, EMB)]
                for c in range(VOCABS[2]):
                    k = (a * VOCABS[1] + b) * VOCABS[2] + c
                    table_v[pl.ds(k * EMB, EMB)] = ab + w_v[pl.ds(off2 + c * EMB, EMB)]

        def in_copy(ii, s):
            cid = wid + ii * NW
            return pltpu.make_async_copy(
                edge_hbm.at[pl.ds(cid * CHUNK, CHUNK)],
                in_v.at[pl.ds(s * CHUNK, CHUNK)],
                in_sems[s])

        def out_copy(ii, s, dt):
            cid = wid + ii * NW
            return pltpu.make_async_copy(
                out_v.at[pl.ds(s * OUT_W + dt * (CHUNKT * TILE_W),
                               CHUNKT * TILE_W)],
                out_hbm.at[dt, pl.ds(cid * CHUNKT * TILE_W, CHUNKT * TILE_W)],
                out_sems[s])

        def compute(s):
            ibase = s * CHUNK
            obase0 = s * OUT_W

            @pl.loop(0, CHUNK // LANES)
            def _grp(j):
                v = in_v[pl.ds(ibase + j * LANES, LANES)]
                a = v & 0xFF
                b = (v >> 8) & 0xFF
                c = (v >> 16) & 0xFF
                code16 = ((a * VOCABS[1] + b) * VOCABS[2] + c) * EMB
                obase = obase0 + (j >> 3) * TILE_W + (j & 7) * LANES
                for d in range(EMB):
                    vv = plsc.load_gather(table_v, [code16 + d])
                    dt, ds = d // 8, d % 8
                    off = obase + dt * (CHUNKT * TILE_W) + ds * 128
                    out_v[pl.ds(off, LANES)] = vv

        in_copy(0, 0).start()

        @pl.when(nw_chunks > 1)
        def _():
            in_copy(1, 1).start()

        @pl.loop(0, niter, step=2)
        def _it(i):
            for s in range(2):
                ii = i + s

                @pl.when(ii < nw_chunks)
                def _():
                    in_copy(ii, s).wait()

                    @pl.when(ii >= 2)
                    def _():
                        out_copy(ii - 2, s, 0).wait()
                        out_copy(ii - 2, s, 1).wait()

                    compute(s)
                    out_copy(ii, s, 0).start()
                    out_copy(ii, s, 1).start()

                    @pl.when(ii + 2 < nw_chunks)
                    def _():
                        in_copy(ii + 2, s).start()

        for s in range(2):
            last = nw_chunks - 1 - (nw_chunks - 1 + s) % 2
            out_copy(last, s, 0).wait()
            out_copy(last, s, 1).wait()

    def run(edge_packed, w_packed):
        x = body(edge_packed, w_packed)
        return x.reshape(2, ntile, 8, 128).transpose(1, 3, 0, 2).reshape(E, EMB)

    return run


def kernel(edge_attr, W0, W1, W2):
    E = edge_attr.shape[0]
    edge8 = jnp.concatenate(
        [edge_attr.astype(jnp.int8), jnp.zeros((E, 1), jnp.int8)], axis=1)
    edge_packed = lax.bitcast_convert_type(edge8, jnp.int32)
    w_packed = jnp.concatenate(
        [W0.reshape(-1), W1.reshape(-1), W2.reshape(-1)])
    return _build_sc_kernel(E)(edge_packed, w_packed)


# speedup vs baseline: 24.0002x; 1.9470x over previous
import functools

import jax
import jax.numpy as jnp
from jax import lax
from jax.experimental import pallas as pl
from jax.experimental.pallas import tpu as pltpu
from jax.experimental.pallas import tpu_sc as plsc

EMB = 16
VOCABS = (5, 6, 2)
NCODES = VOCABS[0] * VOCABS[1] * VOCABS[2]
LANES = 16
CHUNKT = 20
CHUNK = CHUNKT * 128
TILE_W = 8 * 128
OUT_W = 2 * CHUNKT * TILE_W


@functools.cache
def _build_sc_kernel(E: int):
    NC, NS = 2, 16
    NW = NC * NS
    ntile = E // 128
    assert E % 128 == 0 and ntile % CHUNKT == 0
    nchunk = ntile // CHUNKT
    niter = -(-nchunk // NW)
    niter += niter % 2

    mesh = plsc.VectorSubcoreMesh(core_axis_name="c", subcore_axis_name="s")

    @functools.partial(
        pl.kernel,
        out_type=jax.ShapeDtypeStruct((2, ntile * TILE_W), jnp.float32),
        mesh=mesh,
        compiler_params=pltpu.CompilerParams(
            needs_layout_passes=False, use_tc_tiling_on_sc=False),
        scratch_types=[
            pltpu.VMEM((13 * EMB,), jnp.float32),
            pltpu.VMEM((NCODES * EMB,), jnp.float32),
            pltpu.VMEM((2 * CHUNK,), jnp.int32),
            pltpu.VMEM((2 * OUT_W,), jnp.float32),
            pltpu.SemaphoreType.DMA,
            pltpu.SemaphoreType.DMA,
            pltpu.SemaphoreType.DMA,
            pltpu.SemaphoreType.DMA,
        ],
    )
    def body(edge_hbm, w_hbm, out_hbm, w_v, table_v, in_v, out_v,
             in_sem0, in_sem1, out_sem0, out_sem1):
        wid = lax.axis_index("s") * NC + lax.axis_index("c")
        in_sems = (in_sem0, in_sem1)
        out_sems = (out_sem0, out_sem1)
        nw_chunks = (nchunk - wid + NW - 1) // NW

        pltpu.sync_copy(w_hbm, w_v)
        off1 = VOCABS[0] * EMB
        off2 = (VOCABS[0] + VOCABS[1]) * EMB
        for a in range(VOCABS[0]):
            for b in range(VOCABS[1]):
                ab = w_v[pl.ds(a * EMB, EMB)] + w_v[pl.ds(off1 + b * EMB, EMB)]
                for c in range(VOCABS[2]):
                    k = (a * VOCABS[1] + b) * VOCABS[2] + c
                    table_v[pl.ds(k * EMB, EMB)] = ab + w_v[pl.ds(off2 + c * EMB, EMB)]

        def in_copy(ii, s):
            cid = wid + ii * NW
            return pltpu.make_async_copy(
                edge_hbm.at[pl.ds(cid * CHUNK, CHUNK)],
                in_v.at[pl.ds(s * CHUNK, CHUNK)],
                in_sems[s])

        def out_copy(ii, s, dt):
            cid = wid + ii * NW
            return pltpu.make_async_copy(
                out_v.at[pl.ds(s * OUT_W + dt * (CHUNKT * TILE_W),
                               CHUNKT * TILE_W)],
                out_hbm.at[dt, pl.ds(cid * CHUNKT * TILE_W, CHUNKT * TILE_W)],
                out_sems[s])

        def compute(s):
            ibase = s * CHUNK
            obase0 = s * OUT_W

            @pl.loop(0, CHUNK // LANES)
            def _grp(j):
                v = in_v[pl.ds(ibase + j * LANES, LANES)]
                a = v & 0xFF
                b = (v >> 8) & 0xFF
                c = (v >> 16) & 0xFF
                code16 = ((a * VOCABS[1] + b) * VOCABS[2] + c) * EMB
                obase = obase0 + (j >> 3) * TILE_W + (j & 7) * LANES
                vals = [plsc.load_gather(table_v, [code16 + d])
                        for d in range(EMB)]
                for d in range(EMB):
                    dt, ds = d // 8, d % 8
                    off = obase + dt * (CHUNKT * TILE_W) + ds * 128
                    out_v[pl.ds(off, LANES)] = vals[d]

        in_copy(0, 0).start()

        @pl.when(nw_chunks > 1)
        def _():
            in_copy(1, 1).start()

        @pl.loop(0, niter, step=2)
        def _it(i):
            for s in range(2):
                ii = i + s

                @pl.when(ii < nw_chunks)
                def _():
                    in_copy(ii, s).wait()

                    @pl.when(ii >= 2)
                    def _():
                        out_copy(ii - 2, s, 0).wait()
                        out_copy(ii - 2, s, 1).wait()

                    compute(s)
                    out_copy(ii, s, 0).start()
                    out_copy(ii, s, 1).start()

                    @pl.when(ii + 2 < nw_chunks)
                    def _():
                        in_copy(ii + 2, s).start()

        for s in range(2):
            last = nw_chunks - 1 - (nw_chunks - 1 + s) % 2
            out_copy(last, s, 0).wait()
            out_copy(last, s, 1).wait()

    def run(edge_packed, w_packed):
        x = body(edge_packed, w_packed)
        return x.reshape(2, ntile, 8, 128).transpose(1, 3, 0, 2).reshape(E, EMB)

    return run


def kernel(edge_attr, W0, W1, W2):
    E = edge_attr.shape[0]
    edge8 = jnp.concatenate(
        [edge_attr.astype(jnp.int8), jnp.zeros((E, 1), jnp.int8)], axis=1)
    edge_packed = lax.bitcast_convert_type(edge8, jnp.int32)
    w_packed = jnp.concatenate(
        [W0.reshape(-1), W1.reshape(-1), W2.reshape(-1)])
    return _build_sc_kernel(E)(edge_packed, w_packed)


# speedup vs baseline: 24.0494x; 1.0021x over previous
import functools

import jax
import jax.numpy as jnp
from jax import lax
from jax.experimental import pallas as pl
from jax.experimental.pallas import tpu as pltpu
from jax.experimental.pallas import tpu_sc as plsc

EMB = 16
VOCABS = (5, 6, 2)
NCODES = VOCABS[0] * VOCABS[1] * VOCABS[2]
LANES = 16
CHUNKT = 20
CHUNK = CHUNKT * 128
TILE_W = 8 * 128
OUT_W = 2 * CHUNKT * TILE_W


@functools.cache
def _build_sc_kernel(E: int):
    NC, NS = 2, 16
    NW = NC * NS
    ntile = E // 128
    assert E % 128 == 0 and ntile % CHUNKT == 0
    nchunk = ntile // CHUNKT
    niter = -(-nchunk // NW)
    niter += niter % 2

    mesh = plsc.VectorSubcoreMesh(core_axis_name="c", subcore_axis_name="s")

    @functools.partial(
        pl.kernel,
        out_type=jax.ShapeDtypeStruct((2, ntile * TILE_W), jnp.float32),
        mesh=mesh,
        compiler_params=pltpu.CompilerParams(
            needs_layout_passes=False, use_tc_tiling_on_sc=False),
        scratch_types=[
            pltpu.VMEM((13 * EMB,), jnp.float32),
            pltpu.VMEM((NCODES * EMB,), jnp.float32),
            pltpu.VMEM((2 * CHUNK,), jnp.int32),
            pltpu.VMEM((2 * OUT_W,), jnp.float32),
            pltpu.SemaphoreType.DMA,
            pltpu.SemaphoreType.DMA,
            pltpu.SemaphoreType.DMA,
            pltpu.SemaphoreType.DMA,
        ],
    )
    def body(edge_hbm, w_hbm, out_hbm, w_v, table_v, in_v, out_v,
             in_sem0, in_sem1, out_sem0, out_sem1):
        wid = lax.axis_index("s") * NC + lax.axis_index("c")
        in_sems = (in_sem0, in_sem1)
        out_sems = (out_sem0, out_sem1)
        nw_chunks = (nchunk - wid + NW - 1) // NW

        pltpu.sync_copy(w_hbm, w_v)
        off1 = VOCABS[0] * EMB
        off2 = (VOCABS[0] + VOCABS[1]) * EMB
        for a in range(VOCABS[0]):
            for b in range(VOCABS[1]):
                ab = w_v[pl.ds(a * EMB, EMB)] + w_v[pl.ds(off1 + b * EMB, EMB)]
                for c in range(VOCABS[2]):
                    k = (a * VOCABS[1] + b) * VOCABS[2] + c
                    table_v[pl.ds(k * EMB, EMB)] = ab + w_v[pl.ds(off2 + c * EMB, EMB)]

        def in_copy(ii, s):
            cid = wid + ii * NW
            return pltpu.make_async_copy(
                edge_hbm.at[pl.ds(cid * CHUNK, CHUNK)],
                in_v.at[pl.ds(s * CHUNK, CHUNK)],
                in_sems[s])

        def out_copy(ii, s, dt):
            cid = wid + ii * NW
            return pltpu.make_async_copy(
                out_v.at[pl.ds(s * OUT_W + dt * (CHUNKT * TILE_W),
                               CHUNKT * TILE_W)],
                out_hbm.at[dt, pl.ds(cid * CHUNKT * TILE_W, CHUNKT * TILE_W)],
                out_sems[s])

        def compute(s):
            ibase = s * CHUNK
            obase0 = s * OUT_W

            @pl.loop(0, CHUNK // LANES, unroll=2)
            def _grp(j):
                v = in_v[pl.ds(ibase + j * LANES, LANES)]
                a = v & 0xFF
                b = (v >> 8) & 0xFF
                c = (v >> 16) & 0xFF
                code16 = ((a * VOCABS[1] + b) * VOCABS[2] + c) * EMB
                obase = obase0 + (j >> 3) * TILE_W + (j & 7) * LANES
                vals = [plsc.load_gather(table_v, [code16 + d])
                        for d in range(EMB)]
                for d in range(EMB):
                    dt, ds = d // 8, d % 8
                    off = obase + dt * (CHUNKT * TILE_W) + ds * 128
                    out_v[pl.ds(off, LANES)] = vals[d]

        in_copy(0, 0).start()

        @pl.when(nw_chunks > 1)
        def _():
            in_copy(1, 1).start()

        @pl.loop(0, niter, step=2)
        def _it(i):
            for s in range(2):
                ii = i + s

                @pl.when(ii < nw_chunks)
                def _():
                    in_copy(ii, s).wait()

                    @pl.when(ii >= 2)
                    def _():
                        out_copy(ii - 2, s, 0).wait()
                        out_copy(ii - 2, s, 1).wait()

                    compute(s)
                    out_copy(ii, s, 0).start()
                    out_copy(ii, s, 1).start()

                    @pl.when(ii + 2 < nw_chunks)
                    def _():
                        in_copy(ii + 2, s).start()

        for s in range(2):
            last = nw_chunks - 1 - (nw_chunks - 1 + s) % 2
            out_copy(last, s, 0).wait()
            out_copy(last, s, 1).wait()

    def run(edge_packed, w_packed):
        x = body(edge_packed, w_packed)
        return x.reshape(2, ntile, 8, 128).transpose(1, 3, 0, 2).reshape(E, EMB)

    return run


def kernel(edge_attr, W0, W1, W2):
    E = edge_attr.shape[0]
    edge8 = jnp.concatenate(
        [edge_attr.astype(jnp.int8), jnp.zeros((E, 1), jnp.int8)], axis=1)
    edge_packed = lax.bitcast_convert_type(edge8, jnp.int32)
    w_packed = jnp.concatenate(
        [W0.reshape(-1), W1.reshape(-1), W2.reshape(-1)])
    return _build_sc_kernel(E)(edge_packed, w_packed)


# speedup vs baseline: 37.3285x; 1.5522x over previous
import functools

import jax
import jax.numpy as jnp
from jax import lax
from jax.experimental import pallas as pl
from jax.experimental.pallas import tpu as pltpu
from jax.experimental.pallas import tpu_sc as plsc

EMB = 16
VOCABS = (5, 6, 2)
NCODES = VOCABS[0] * VOCABS[1] * VOCABS[2]
LANES = 16
CHUNKT = 20
CHUNK = CHUNKT * 128
TILE_W = 8 * 128
TSTRIDE = 17
OUT_W = 2 * CHUNKT * TILE_W


@functools.cache
def _build_sc_kernel(E: int):
    NC, NS = 2, 16
    NW = NC * NS
    ntile = E // 128
    assert E % 128 == 0 and ntile % CHUNKT == 0
    nchunk = ntile // CHUNKT
    niter = -(-nchunk // NW)
    niter += niter % 2

    mesh = plsc.VectorSubcoreMesh(core_axis_name="c", subcore_axis_name="s")

    @functools.partial(
        pl.kernel,
        out_type=jax.ShapeDtypeStruct((2, ntile * TILE_W), jnp.float32),
        mesh=mesh,
        compiler_params=pltpu.CompilerParams(
            needs_layout_passes=False, use_tc_tiling_on_sc=False),
        scratch_types=[
            pltpu.VMEM((13 * EMB,), jnp.float32),
            pltpu.VMEM((NCODES * TSTRIDE,), jnp.float32),
            pltpu.VMEM((2 * CHUNK,), jnp.int32),
            pltpu.VMEM((2 * OUT_W,), jnp.float32),
            pltpu.SemaphoreType.DMA,
            pltpu.SemaphoreType.DMA,
            pltpu.SemaphoreType.DMA,
            pltpu.SemaphoreType.DMA,
        ],
    )
    def body(edge_hbm, w_hbm, out_hbm, w_v, table_v, in_v, out_v,
             in_sem0, in_sem1, out_sem0, out_sem1):
        wid = lax.axis_index("s") * NC + lax.axis_index("c")
        in_sems = (in_sem0, in_sem1)
        out_sems = (out_sem0, out_sem1)
        nw_chunks = (nchunk - wid + NW - 1) // NW

        pltpu.sync_copy(w_hbm, w_v)
        off1 = VOCABS[0] * EMB
        off2 = (VOCABS[0] + VOCABS[1]) * EMB
        for a in range(VOCABS[0]):
            for b in range(VOCABS[1]):
                ab = w_v[pl.ds(a * EMB, EMB)] + w_v[pl.ds(off1 + b * EMB, EMB)]
                for c in range(VOCABS[2]):
                    k = (a * VOCABS[1] + b) * VOCABS[2] + c
                    table_v[pl.ds(k * TSTRIDE, EMB)] = ab + w_v[pl.ds(off2 + c * EMB, EMB)]

        def in_copy(ii, s):
            cid = wid + ii * NW
            return pltpu.make_async_copy(
                edge_hbm.at[pl.ds(cid * CHUNK, CHUNK)],
                in_v.at[pl.ds(s * CHUNK, CHUNK)],
                in_sems[s])

        def out_copy(ii, s, dt):
            cid = wid + ii * NW
            return pltpu.make_async_copy(
                out_v.at[pl.ds(s * OUT_W + dt * (CHUNKT * TILE_W),
                               CHUNKT * TILE_W)],
                out_hbm.at[dt, pl.ds(cid * CHUNKT * TILE_W, CHUNKT * TILE_W)],
                out_sems[s])

        def compute(s):
            ibase = s * CHUNK
            obase0 = s * OUT_W

            @pl.loop(0, CHUNK // LANES, unroll=2)
            def _grp(j):
                v = in_v[pl.ds(ibase + j * LANES, LANES)]
                a = v & 0xFF
                b = (v >> 8) & 0xFF
                c = (v >> 16) & 0xFF
                code16 = ((a * VOCABS[1] + b) * VOCABS[2] + c) * TSTRIDE
                obase = obase0 + (j >> 3) * TILE_W + (j & 7) * LANES
                vals = [plsc.load_gather(table_v, [code16 + d])
                        for d in range(EMB)]
                for d in range(EMB):
                    dt, ds = d // 8, d % 8
                    off = obase + dt * (CHUNKT * TILE_W) + ds * 128
                    out_v[pl.ds(off, LANES)] = vals[d]

        in_copy(0, 0).start()

        @pl.when(nw_chunks > 1)
        def _():
            in_copy(1, 1).start()

        @pl.loop(0, niter, step=2)
        def _it(i):
            for s in range(2):
                ii = i + s

                @pl.when(ii < nw_chunks)
                def _():
                    in_copy(ii, s).wait()

                    @pl.when(ii >= 2)
                    def _():
                        out_copy(ii - 2, s, 0).wait()
                        out_copy(ii - 2, s, 1).wait()

                    compute(s)
                    out_copy(ii, s, 0).start()
                    out_copy(ii, s, 1).start()

                    @pl.when(ii + 2 < nw_chunks)
                    def _():
                        in_copy(ii + 2, s).start()

        for s in range(2):
            last = nw_chunks - 1 - (nw_chunks - 1 + s) % 2
            out_copy(last, s, 0).wait()
            out_copy(last, s, 1).wait()

    def run(edge_packed, w_packed):
        x = body(edge_packed, w_packed)
        return x.reshape(2, ntile, 8, 128).transpose(1, 3, 0, 2).reshape(E, EMB)

    return run


def kernel(edge_attr, W0, W1, W2):
    E = edge_attr.shape[0]
    edge8 = jnp.concatenate(
        [edge_attr.astype(jnp.int8), jnp.zeros((E, 1), jnp.int8)], axis=1)
    edge_packed = lax.bitcast_convert_type(edge8, jnp.int32)
    w_packed = jnp.concatenate(
        [W0.reshape(-1), W1.reshape(-1), W2.reshape(-1)])
    return _build_sc_kernel(E)(edge_packed, w_packed)


# speedup vs baseline: 54.0897x; 1.4490x over previous
import functools

import jax
import jax.numpy as jnp
from jax import lax
from jax.experimental import pallas as pl
from jax.experimental.pallas import tpu as pltpu
from jax.experimental.pallas import tpu_sc as plsc

EMB = 16
VOCABS = (5, 6, 2)
NCODES = VOCABS[0] * VOCABS[1] * VOCABS[2]
LANES = 16
CHUNKT = 20
CHUNK = CHUNKT * 128
TILE_W = 8 * 128
TSTRIDE = 17
OUT_W = 2 * CHUNKT * TILE_W


@functools.cache
def _build_sc_kernel(E: int):
    NC, NS = 2, 16
    NW = NC * NS
    ntile = E // 128
    assert E % 128 == 0 and ntile % CHUNKT == 0
    nchunk = ntile // CHUNKT
    niter = -(-nchunk // NW)
    niter += niter % 2

    mesh = plsc.VectorSubcoreMesh(core_axis_name="c", subcore_axis_name="s")

    @functools.partial(
        pl.kernel,
        out_type=jax.ShapeDtypeStruct((2, ntile * TILE_W), jnp.float32),
        mesh=mesh,
        compiler_params=pltpu.CompilerParams(
            needs_layout_passes=False, use_tc_tiling_on_sc=False),
        scratch_types=[
            pltpu.VMEM((13 * EMB,), jnp.float32),
            pltpu.VMEM((NCODES * TSTRIDE,), jnp.float32),
            pltpu.VMEM((2 * CHUNK,), jnp.int32),
            pltpu.VMEM((2 * OUT_W,), jnp.float32),
            pltpu.SemaphoreType.DMA,
            pltpu.SemaphoreType.DMA,
            pltpu.SemaphoreType.DMA,
            pltpu.SemaphoreType.DMA,
        ],
    )
    def body(edge_hbm, w_hbm, out_hbm, w_v, table_v, in_v, out_v,
             in_sem0, in_sem1, out_sem0, out_sem1):
        wid = lax.axis_index("s") * NC + lax.axis_index("c")
        in_sems = (in_sem0, in_sem1)
        out_sems = (out_sem0, out_sem1)
        nw_chunks = (nchunk - wid + NW - 1) // NW

        pltpu.sync_copy(w_hbm, w_v)
        off1 = VOCABS[0] * EMB
        off2 = (VOCABS[0] + VOCABS[1]) * EMB
        for a in range(VOCABS[0]):
            for b in range(VOCABS[1]):
                ab = w_v[pl.ds(a * EMB, EMB)] + w_v[pl.ds(off1 + b * EMB, EMB)]
                for c in range(VOCABS[2]):
                    k = (a * VOCABS[1] + b) * VOCABS[2] + c
                    table_v[pl.ds(k * TSTRIDE, EMB)] = ab + w_v[pl.ds(off2 + c * EMB, EMB)]

        def in_copy(ii, s):
            cid = wid + ii * NW
            return pltpu.make_async_copy(
                edge_hbm.at[pl.ds(cid * CHUNK, CHUNK)],
                in_v.at[pl.ds(s * CHUNK, CHUNK)],
                in_sems[s])

        def out_copy(ii, s, dt):
            cid = wid + ii * NW
            return pltpu.make_async_copy(
                out_v.at[pl.ds(s * OUT_W + dt * (CHUNKT * TILE_W),
                               CHUNKT * TILE_W)],
                out_hbm.at[dt, pl.ds(cid * CHUNKT * TILE_W, CHUNKT * TILE_W)],
                out_sems[s])

        def compute(s):
            ibase = s * CHUNK
            obase0 = s * OUT_W

            @pl.loop(0, CHUNK // LANES, unroll=2)
            def _grp(j):
                v = in_v[pl.ds(ibase + j * LANES, LANES)]
                a = v & 0xFF
                b = (v >> 8) & 0xFF
                c = (v >> 16) & 0xFF
                code16 = ((a * VOCABS[1] + b) * VOCABS[2] + c) * TSTRIDE
                obase = obase0 + (j >> 3) * TILE_W + (j & 7) * LANES
                vals = [plsc.load_gather(table_v, [code16 + d])
                        for d in range(EMB)]
                for d in range(EMB):
                    dt, ds = d // 8, d % 8
                    off = obase + dt * (CHUNKT * TILE_W) + ds * 128
                    out_v[pl.ds(off, LANES)] = vals[d]

        in_copy(0, 0).start()

        @pl.when(nw_chunks > 1)
        def _():
            in_copy(1, 1).start()

        @pl.loop(0, niter, step=2)
        def _it(i):
            for s in range(2):
                ii = i + s

                @pl.when(ii < nw_chunks)
                def _():
                    in_copy(ii, s).wait()

                    @pl.when(ii >= 2)
                    def _():
                        out_copy(ii - 2, s, 0).wait()
                        out_copy(ii - 2, s, 1).wait()

                    compute(s)
                    out_copy(ii, s, 0).start()
                    out_copy(ii, s, 1).start()

                    @pl.when(ii + 2 < nw_chunks)
                    def _():
                        in_copy(ii + 2, s).start()

        for s in range(2):
            last = nw_chunks - 1 - (nw_chunks - 1 + s) % 2
            out_copy(last, s, 0).wait()
            out_copy(last, s, 1).wait()

    def run(edge_packed, w_packed):
        x = body(edge_packed, w_packed)
        return x.reshape(2, ntile, 8, 128).transpose(1, 3, 0, 2).reshape(E, EMB)

    return run


def kernel(edge_attr, W0, W1, W2):
    E = edge_attr.shape[0]
    et = edge_attr.T
    edge_packed = et[0] | (et[1] << 8) | (et[2] << 16)
    w_packed = jnp.concatenate(
        [W0.reshape(-1), W1.reshape(-1), W2.reshape(-1)])
    return _build_sc_kernel(E)(edge_packed, w_packed)


# speedup vs baseline: 88.2244x; 1.6311x over previous
import functools

import jax
import jax.numpy as jnp
from jax import lax
from jax.experimental import pallas as pl
from jax.experimental.pallas import tpu as pltpu
from jax.experimental.pallas import tpu_sc as plsc

EMB = 16
VOCABS = (5, 6, 2)
NCODES = VOCABS[0] * VOCABS[1] * VOCABS[2]
LANES = 16
CHUNKT = 20
CHUNK = CHUNKT * 128
TILE_W = 8 * 128
TSTRIDE = 17
OUT_W = 2 * CHUNKT * TILE_W


@functools.cache
def _build_sc_kernel(E: int):
    NC, NS = 2, 16
    NW = NC * NS
    ntile = E // 128
    assert E % 128 == 0 and ntile % CHUNKT == 0
    nchunk = ntile // CHUNKT
    niter = -(-nchunk // NW)
    niter += niter % 2

    mesh = plsc.VectorSubcoreMesh(core_axis_name="c", subcore_axis_name="s")

    @functools.partial(
        pl.kernel,
        out_type=jax.ShapeDtypeStruct((2, ntile * TILE_W), jnp.float32),
        mesh=mesh,
        compiler_params=pltpu.CompilerParams(
            needs_layout_passes=False, use_tc_tiling_on_sc=False),
        scratch_types=[
            pltpu.VMEM((13 * EMB,), jnp.float32),
            pltpu.VMEM((NCODES * TSTRIDE,), jnp.float32),
            pltpu.VMEM((2 * CHUNK,), jnp.int32),
            pltpu.VMEM((2 * OUT_W,), jnp.float32),
            pltpu.SemaphoreType.DMA,
            pltpu.SemaphoreType.DMA,
            pltpu.SemaphoreType.DMA,
            pltpu.SemaphoreType.DMA,
        ],
    )
    def body(edge_hbm, w_hbm, out_hbm, w_v, table_v, in_v, out_v,
             in_sem0, in_sem1, out_sem0, out_sem1):
        wid = lax.axis_index("s") * NC + lax.axis_index("c")
        in_sems = (in_sem0, in_sem1)
        out_sems = (out_sem0, out_sem1)
        nw_chunks = (nchunk - wid + NW - 1) // NW

        pltpu.sync_copy(w_hbm, w_v)
        off1 = VOCABS[0] * EMB
        off2 = (VOCABS[0] + VOCABS[1]) * EMB
        for a in range(VOCABS[0]):
            for b in range(VOCABS[1]):
                ab = w_v[pl.ds(a * EMB, EMB)] + w_v[pl.ds(off1 + b * EMB, EMB)]
                for c in range(VOCABS[2]):
                    k = (a * VOCABS[1] + b) * VOCABS[2] + c
                    table_v[pl.ds(k * TSTRIDE, EMB)] = ab + w_v[pl.ds(off2 + c * EMB, EMB)]

        def in_copy(ii, s):
            cid = wid + ii * NW
            return pltpu.make_async_copy(
                edge_hbm.at[pl.ds(cid * CHUNK, CHUNK)],
                in_v.at[pl.ds(s * CHUNK, CHUNK)],
                in_sems[s])

        def out_copy(ii, s, dt):
            cid = wid + ii * NW
            return pltpu.make_async_copy(
                out_v.at[pl.ds(s * OUT_W + dt * (CHUNKT * TILE_W),
                               CHUNKT * TILE_W)],
                out_hbm.at[dt, pl.ds(cid * CHUNKT * TILE_W, CHUNKT * TILE_W)],
                out_sems[s])

        def compute(s):
            ibase = s * CHUNK
            obase0 = s * OUT_W

            @plsc.parallel_loop(0, CHUNK // LANES, unroll=2)
            def _grp(j):
                v = in_v[pl.ds(ibase + j * LANES, LANES)]
                a = v & 0xFF
                b = (v >> 8) & 0xFF
                c = (v >> 16) & 0xFF
                code16 = ((a * VOCABS[1] + b) * VOCABS[2] + c) * TSTRIDE
                obase = obase0 + (j >> 3) * TILE_W + (j & 7) * LANES
                vals = [plsc.load_gather(table_v, [code16 + d])
                        for d in range(EMB)]
                for d in range(EMB):
                    dt, ds = d // 8, d % 8
                    off = obase + dt * (CHUNKT * TILE_W) + ds * 128
                    out_v[pl.ds(off, LANES)] = vals[d]

        in_copy(0, 0).start()

        @pl.when(nw_chunks > 1)
        def _():
            in_copy(1, 1).start()

        @pl.loop(0, niter, step=2)
        def _it(i):
            for s in range(2):
                ii = i + s

                @pl.when(ii < nw_chunks)
                def _():
                    in_copy(ii, s).wait()

                    @pl.when(ii >= 2)
                    def _():
                        out_copy(ii - 2, s, 0).wait()
                        out_copy(ii - 2, s, 1).wait()

                    compute(s)
                    out_copy(ii, s, 0).start()
                    out_copy(ii, s, 1).start()

                    @pl.when(ii + 2 < nw_chunks)
                    def _():
                        in_copy(ii + 2, s).start()

        for s in range(2):
            last = nw_chunks - 1 - (nw_chunks - 1 + s) % 2
            out_copy(last, s, 0).wait()
            out_copy(last, s, 1).wait()

    def run(edge_packed, w_packed):
        x = body(edge_packed, w_packed)
        return x.reshape(2, ntile, 8, 128).transpose(1, 3, 0, 2).reshape(E, EMB)

    return run


def kernel(edge_attr, W0, W1, W2):
    E = edge_attr.shape[0]
    et = edge_attr.T
    edge_packed = et[0] | (et[1] << 8) | (et[2] << 16)
    w_packed = jnp.concatenate(
        [W0.reshape(-1), W1.reshape(-1), W2.reshape(-1)])
    return _build_sc_kernel(E)(edge_packed, w_packed)
